# Initial kernel scaffold; baseline (speedup 1.0000x reference)
import jax
import jax.numpy as jnp
from jax.experimental import pallas as pl


def kernel(x, edge_index, batch, W1, b1, W2, b2, W3, b3, s1, s2, s3, Wl1, bl1, Wl2, bl2, Wl3, bl3):
    raise NotImplementedError("write your pallas kernel here")


# speedup vs baseline: 7.7290x; 7.7290x over previous
import functools

import jax
import jax.numpy as jnp
from jax import lax
from jax.experimental import pallas as pl
from jax.experimental.pallas import tpu as pltpu
from jax.experimental.pallas import tpu_sc as plsc

_N = 10000
_E = 320000
_H = 128
_B = 64
_C = 10

_NC = 2
_NS = 16
_NW = _NC * _NS
_EPW = _E // _NW
_K = 80
_NBATCH = _EPW // _K
_NPAD = 10240
_RPW = _NPAD // _NS
_RCH = 128
_NCH = _RPW // _RCH

_BLK = 1000
_NBLK = _N // _BLK

_sc_mesh = plsc.VectorSubcoreMesh(core_axis_name="c", subcore_axis_name="s")


@functools.partial(
    pl.kernel,
    mesh=_sc_mesh,
    out_type=jax.ShapeDtypeStruct((_NC, _NPAD, _H), jnp.float32),
    scratch_types=[
        pltpu.VMEM((_NBATCH, _K), jnp.int32),
        pltpu.VMEM((_K, _H), jnp.float32),
        pltpu.VMEM((_RCH,), jnp.int32),
        pltpu.VMEM((_RCH, _H), jnp.float32),
        pltpu.VMEM_SHARED((_NPAD, _H), jnp.float32),
        pltpu.SemaphoreType.DMA,
    ],
)
def _sc_degree(dst_hbm, out_hbm, dstb_v, ones_v, ridx_v, stage_v, acc_sh, sem):
    c = lax.axis_index("c")
    s = lax.axis_index("s")
    w = c * _NS + s

    def fill_ones(i, carry):
        for j in range(_H // 16):
            ones_v[i, pl.ds(j * 16, 16)] = jnp.full((16,), 1.0, jnp.float32)
        return carry

    lax.fori_loop(0, _K, fill_ones, 0)

    def fill_zero(i, carry):
        for j in range(_H // 16):
            stage_v[i, pl.ds(j * 16, 16)] = jnp.zeros((16,), jnp.float32)
        return carry

    lax.fori_loop(0, _RCH, fill_zero, 0)

    base_row = s * _RPW
    for t in range(_NCH):
        def fill_idx(i, carry):
            ridx_v[pl.ds(i * 16, 16)] = (base_row + t * _RCH + i * 16
                                         + lax.iota(jnp.int32, 16))
            return carry

        lax.fori_loop(0, _RCH // 16, fill_idx, 0)
        pltpu.sync_copy(stage_v, acc_sh.at[ridx_v])
    plsc.subcore_barrier()

    pltpu.sync_copy(dst_hbm.at[w], dstb_v)

    def body(i, carry):
        pltpu.sync_copy(ones_v, acc_sh.at[dstb_v.at[i]], add=True)
        return carry

    lax.fori_loop(0, _NBATCH, body, 0)
    plsc.subcore_barrier()

    for t in range(_NCH):
        def fill_idx2(i, carry):
            ridx_v[pl.ds(i * 16, 16)] = (base_row + t * _RCH + i * 16
                                         + lax.iota(jnp.int32, 16))
            return carry

        lax.fori_loop(0, _RCH // 16, fill_idx2, 0)
        pltpu.async_copy(acc_sh.at[ridx_v], stage_v, sem).wait()
        pltpu.sync_copy(stage_v, out_hbm.at[c, pl.ds(base_row + t * _RCH, _RCH)])


@functools.partial(
    pl.kernel,
    mesh=_sc_mesh,
    out_type=jax.ShapeDtypeStruct((_NC, _NPAD, _H), jnp.float32),
    scratch_types=[
        pltpu.VMEM((1, _K), jnp.int32),
        pltpu.VMEM((1, _K), jnp.int32),
        pltpu.VMEM((_K, _H), jnp.float32),
        pltpu.VMEM((_RCH,), jnp.int32),
        pltpu.VMEM((_RCH, _H), jnp.float32),
        pltpu.VMEM_SHARED((_NPAD, _H), jnp.float32),
        pltpu.SemaphoreType.DMA,
    ],
)
def _sc_aggregate(g_hbm, src_hbm, dst_hbm, out_hbm,
                  srcb_v, dstb_v, rows_v, ridx_v, stage_v, acc_sh, sem):
    c = lax.axis_index("c")
    s = lax.axis_index("s")
    w = c * _NS + s

    def fill_zero(i, carry):
        for j in range(_H // 16):
            stage_v[i, pl.ds(j * 16, 16)] = jnp.zeros((16,), jnp.float32)
        return carry

    lax.fori_loop(0, _RCH, fill_zero, 0)

    base_row = s * _RPW
    for t in range(_NCH):
        def fill_idx(i, carry):
            ridx_v[pl.ds(i * 16, 16)] = (base_row + t * _RCH + i * 16
                                         + lax.iota(jnp.int32, 16))
            return carry

        lax.fori_loop(0, _RCH // 16, fill_idx, 0)
        pltpu.sync_copy(stage_v, acc_sh.at[ridx_v])
    plsc.subcore_barrier()

    def body(i, carry):
        pltpu.sync_copy(src_hbm.at[w, i], srcb_v.at[0])
        pltpu.sync_copy(dst_hbm.at[w, i], dstb_v.at[0])
        pltpu.async_copy(g_hbm.at[srcb_v.at[0]], rows_v, sem).wait()
        pltpu.sync_copy(rows_v, acc_sh.at[dstb_v.at[0]], add=True)
        return carry

    lax.fori_loop(0, _NBATCH, body, 0)
    plsc.subcore_barrier()

    for t in range(_NCH):
        def fill_idx2(i, carry):
            ridx_v[pl.ds(i * 16, 16)] = (base_row + t * _RCH + i * 16
                                         + lax.iota(jnp.int32, 16))
            return carry

        lax.fori_loop(0, _RCH // 16, fill_idx2, 0)
        pltpu.async_copy(acc_sh.at[ridx_v], stage_v, sem).wait()
        pltpu.sync_copy(stage_v, out_hbm.at[c, pl.ds(base_row + t * _RCH, _RCH)])


def _stage0_body(x_ref, w1_ref, degp_ref, batch_ref, g_ref, inv_ref, cnt_ref):
    deg = degp_ref[0, :, :1] + degp_ref[1, :, :1]
    inv = lax.rsqrt(jnp.maximum(deg, 1.0))
    inv_ref[...] = jnp.broadcast_to(inv, (_BLK, 16))
    g_ref[...] = jnp.dot(x_ref[...], w1_ref[...],
                         preferred_element_type=jnp.float32) * inv
    iota = lax.broadcasted_iota(jnp.int32, (1, _B), 1)
    onehot = (batch_ref[...] == iota).astype(jnp.float32)
    cnt = jnp.sum(onehot, axis=0)

    @pl.when(pl.program_id(0) == 0)
    def _():
        cnt_ref[...] = jnp.zeros((_B, _H), jnp.float32)

    cnt_ref[...] += jnp.broadcast_to(cnt[:, None], (_B, _H))


_stage0 = pl.pallas_call(
    _stage0_body,
    grid=(_NBLK,),
    in_specs=[
        pl.BlockSpec((_BLK, _H), lambda i: (i, 0)),
        pl.BlockSpec((_H, _H), lambda i: (0, 0)),
        pl.BlockSpec((_NC, _BLK, _H), lambda i: (0, i, 0)),
        pl.BlockSpec((_BLK, 1), lambda i: (i, 0)),
    ],
    out_specs=[
        pl.BlockSpec((_BLK, _H), lambda i: (i, 0)),
        pl.BlockSpec((_BLK, 16), lambda i: (i, 0)),
        pl.BlockSpec((_B, _H), lambda i: (0, 0)),
    ],
    out_shape=[
        jax.ShapeDtypeStruct((_N, _H), jnp.float32),
        jax.ShapeDtypeStruct((_N, 16), jnp.float32),
        jax.ShapeDtypeStruct((_B, _H), jnp.float32),
    ],
)


def _layer_body(with_next, *refs):
    if with_next:
        (parts_ref, inv_ref, b_ref, s_ref, batch_ref, wn_ref,
         g_ref, gmp_ref, gsum_ref, scal_ref) = refs
    else:
        (parts_ref, inv_ref, b_ref, s_ref, batch_ref,
         gmp_ref, gsum_ref, scal_ref) = refs

    inv = inv_ref[:, :1]
    h = parts_ref[0] + parts_ref[1]
    h = jnp.maximum(h * inv + b_ref[...], 0.0)
    z = jnp.sum(h * s_ref[...], axis=1, keepdims=True)
    p = jax.nn.sigmoid(z)
    pc = jnp.clip(p, 1e-6, 1.0 - 1e-6)
    klp = jnp.sum(pc * jnp.log(2.0 * pc)
                  + (1.0 - pc) * jnp.log(2.0 * (1.0 - pc)))
    refp = jnp.sum((pc - 0.5) ** 2)
    hp = h * p

    if with_next:
        g_ref[...] = jnp.dot(hp, wn_ref[...],
                             preferred_element_type=jnp.float32) * inv

    iota = lax.broadcasted_iota(jnp.int32, (1, _B), 1)
    onehot = (batch_ref[...] == iota).astype(jnp.float32)
    gsum_blk = lax.dot_general(onehot, hp, (((0,), (0,)), ((), ())),
                               preferred_element_type=jnp.float32)

    rows = []
    for b in range(_B):
        mask = (batch_ref[...] == b).astype(jnp.float32)
        rows.append(jnp.max(hp * mask, axis=0, keepdims=True))
    gmp_blk = jnp.concatenate(rows, axis=0)

    rowidx = lax.broadcasted_iota(jnp.int32, (8, _H), 0)
    scal_blk = (jnp.where(rowidx == 0, klp, 0.0)
                + jnp.where(rowidx == 1, refp, 0.0))

    @pl.when(pl.program_id(0) == 0)
    def _():
        gmp_ref[...] = jnp.zeros((_B, _H), jnp.float32)
        gsum_ref[...] = jnp.zeros((_B, _H), jnp.float32)
        scal_ref[...] = jnp.zeros((8, _H), jnp.float32)

    gmp_ref[...] = jnp.maximum(gmp_ref[...], gmp_blk)
    gsum_ref[...] += gsum_blk
    scal_ref[...] += scal_blk


_layer_in_specs = [
    pl.BlockSpec((_NC, _BLK, _H), lambda i: (0, i, 0)),
    pl.BlockSpec((_BLK, 16), lambda i: (i, 0)),
    pl.BlockSpec((1, _H), lambda i: (0, 0)),
    pl.BlockSpec((1, _H), lambda i: (0, 0)),
    pl.BlockSpec((_BLK, 1), lambda i: (i, 0)),
]
_readout_out_specs = [
    pl.BlockSpec((_B, _H), lambda i: (0, 0)),
    pl.BlockSpec((_B, _H), lambda i: (0, 0)),
    pl.BlockSpec((8, _H), lambda i: (0, 0)),
]
_readout_out_shape = [
    jax.ShapeDtypeStruct((_B, _H), jnp.float32),
    jax.ShapeDtypeStruct((_B, _H), jnp.float32),
    jax.ShapeDtypeStruct((8, _H), jnp.float32),
]

_layer_mid = pl.pallas_call(
    functools.partial(_layer_body, True),
    grid=(_NBLK,),
    in_specs=_layer_in_specs + [pl.BlockSpec((_H, _H), lambda i: (0, 0))],
    out_specs=[pl.BlockSpec((_BLK, _H), lambda i: (i, 0))] + _readout_out_specs,
    out_shape=[jax.ShapeDtypeStruct((_N, _H), jnp.float32)] + _readout_out_shape,
)

_layer_last = pl.pallas_call(
    functools.partial(_layer_body, False),
    grid=(_NBLK,),
    in_specs=_layer_in_specs,
    out_specs=_readout_out_specs,
    out_shape=_readout_out_shape,
)


def _final_body(gmp1, gsum1, gmp2, gsum2, gmp3, gsum3, cnt,
                wl1, bl1, wl2, bl2, wl3, bl3, out_ref):
    cntc = jnp.maximum(cnt[...], 1.0)

    def xin(gmp, gsum):
        return jnp.concatenate([gmp[...], gsum[...] / cntc], axis=1)

    g = (jnp.maximum(xin(gmp1, gsum1), 0.0)
         + jnp.maximum(xin(gmp2, gsum2), 0.0)
         + jnp.maximum(xin(gmp3, gsum3), 0.0))
    g = jnp.maximum(jnp.dot(g, wl1[...],
                            preferred_element_type=jnp.float32) + bl1[...], 0.0)
    g = jnp.maximum(jnp.dot(g, wl2[...],
                            preferred_element_type=jnp.float32) + bl2[...], 0.0)
    z = jnp.dot(g, wl3[...], preferred_element_type=jnp.float32) + bl3[...]
    lane = lax.broadcasted_iota(jnp.int32, (_B, _H), 1)
    zm = jnp.where(lane < _C, z, -1e30)
    mx = jnp.max(zm, axis=1, keepdims=True)
    lse = mx + jnp.log(jnp.sum(jnp.exp(zm - mx), axis=1, keepdims=True))
    out_ref[...] = zm - lse


_final = pl.pallas_call(
    _final_body,
    grid=(1,),
    in_specs=[pl.BlockSpec((_B, _H), lambda i: (0, 0))] * 7
    + [
        pl.BlockSpec((2 * _H, _H), lambda i: (0, 0)),
        pl.BlockSpec((1, _H), lambda i: (0, 0)),
        pl.BlockSpec((_H, _H), lambda i: (0, 0)),
        pl.BlockSpec((1, _H), lambda i: (0, 0)),
        pl.BlockSpec((_H, _H), lambda i: (0, 0)),
        pl.BlockSpec((1, _H), lambda i: (0, 0)),
    ],
    out_specs=pl.BlockSpec((_B, _H), lambda i: (0, 0)),
    out_shape=jax.ShapeDtypeStruct((_B, _H), jnp.float32),
)


def kernel(x, edge_index, batch, W1, b1, W2, b2, W3, b3,
           s1, s2, s3, Wl1, bl1, Wl2, bl2, Wl3, bl3):
    src3 = edge_index[0].reshape(_NW, _NBATCH, _K)
    dst3 = edge_index[1].reshape(_NW, _NBATCH, _K)
    batch2 = batch.reshape(_N, 1)

    degp = _sc_degree(dst3)
    g1, inv16, cnt = _stage0(x, W1, degp, batch2)

    p1 = _sc_aggregate(g1, src3, dst3)
    g2, gmp1, gsum1, sc1 = _layer_mid(p1, inv16, b1.reshape(1, _H),
                                      s1.reshape(1, _H), batch2, W2)
    p2 = _sc_aggregate(g2, src3, dst3)
    g3, gmp2, gsum2, sc2 = _layer_mid(p2, inv16, b2.reshape(1, _H),
                                      s2.reshape(1, _H), batch2, W3)
    p3 = _sc_aggregate(g3, src3, dst3)
    gmp3, gsum3, sc3 = _layer_last(p3, inv16, b3.reshape(1, _H),
                                   s3.reshape(1, _H), batch2)

    Wl2p = jnp.pad(Wl2, ((0, 0), (0, _H - Wl2.shape[1])))
    bl2p = jnp.pad(bl2, (0, _H - bl2.shape[0])).reshape(1, _H)
    Wl3p = jnp.pad(Wl3, ((0, _H - Wl3.shape[0]), (0, _H - Wl3.shape[1])))
    bl3p = jnp.pad(bl3, (0, _H - bl3.shape[0])).reshape(1, _H)

    logits128 = _final(gmp1, gsum1, gmp2, gsum2, gmp3, gsum3, cnt,
                       Wl1, bl1.reshape(1, _H), Wl2p, bl2p, Wl3p, bl3p)
    logits = logits128[:, :_C]
    kl = (sc1[0, 0] + sc2[0, 0] + sc3[0, 0]) / _N
    refv = (sc1[1, 0] + sc2[1, 0] + sc3[1, 0]) / _N
    return (logits, kl, refv)


# speedup vs baseline: 10.9964x; 1.4227x over previous
import functools

import jax
import jax.numpy as jnp
from jax import lax
from jax.experimental import pallas as pl
from jax.experimental.pallas import tpu as pltpu
from jax.experimental.pallas import tpu_sc as plsc

_N = 10000
_E = 320000
_H = 128
_B = 64
_C = 10

_NC = 2
_NS = 16
_NW = _NC * _NS
_EPW = _E // _NW
_K = 80
_NBATCH = _EPW // _K
_NPAD = 10240
_RPW = _NPAD // _NS
_RCH = 128
_NCH = _RPW // _RCH

_BLK = 1000
_NBLK = _N // _BLK

_sc_mesh = plsc.VectorSubcoreMesh(core_axis_name="c", subcore_axis_name="s")


@functools.partial(
    pl.kernel,
    mesh=_sc_mesh,
    out_type=jax.ShapeDtypeStruct((_NC, _NPAD, _H), jnp.float32),
    scratch_types=[
        pltpu.VMEM((_NBATCH, _K), jnp.int32),
        pltpu.VMEM((_K, _H), jnp.float32),
        pltpu.VMEM((_RCH,), jnp.int32),
        pltpu.VMEM((_RCH, _H), jnp.float32),
        pltpu.VMEM_SHARED((_NPAD, _H), jnp.float32),
        pltpu.SemaphoreType.DMA,
    ],
)
def _sc_degree(dst_hbm, out_hbm, dstb_v, ones_v, ridx_v, stage_v, acc_sh, sem):
    c = lax.axis_index("c")
    s = lax.axis_index("s")
    w = c * _NS + s

    def fill_ones(i, carry):
        for j in range(_H // 16):
            ones_v[i, pl.ds(j * 16, 16)] = jnp.full((16,), 1.0, jnp.float32)
        return carry

    lax.fori_loop(0, _K, fill_ones, 0)

    def fill_zero(i, carry):
        for j in range(_H // 16):
            stage_v[i, pl.ds(j * 16, 16)] = jnp.zeros((16,), jnp.float32)
        return carry

    lax.fori_loop(0, _RCH, fill_zero, 0)

    base_row = s * _RPW
    for t in range(_NCH):
        def fill_idx(i, carry):
            ridx_v[pl.ds(i * 16, 16)] = (base_row + t * _RCH + i * 16
                                         + lax.iota(jnp.int32, 16))
            return carry

        lax.fori_loop(0, _RCH // 16, fill_idx, 0)
        pltpu.sync_copy(stage_v, acc_sh.at[ridx_v])
    plsc.subcore_barrier()

    pltpu.sync_copy(dst_hbm.at[w], dstb_v)

    def body(i, carry):
        pltpu.sync_copy(ones_v, acc_sh.at[dstb_v.at[i]], add=True)
        return carry

    lax.fori_loop(0, _NBATCH, body, 0)
    plsc.subcore_barrier()

    for t in range(_NCH):
        def fill_idx2(i, carry):
            ridx_v[pl.ds(i * 16, 16)] = (base_row + t * _RCH + i * 16
                                         + lax.iota(jnp.int32, 16))
            return carry

        lax.fori_loop(0, _RCH // 16, fill_idx2, 0)
        pltpu.async_copy(acc_sh.at[ridx_v], stage_v, sem).wait()
        pltpu.sync_copy(stage_v, out_hbm.at[c, pl.ds(base_row + t * _RCH, _RCH)])


@functools.partial(
    pl.kernel,
    mesh=_sc_mesh,
    out_type=jax.ShapeDtypeStruct((_NC, _NPAD, _H), jnp.float32),
    scratch_types=[
        pltpu.VMEM((1, _K), jnp.int32),
        pltpu.VMEM((1, _K), jnp.int32),
        pltpu.VMEM((1, _K), jnp.int32),
        pltpu.VMEM((1, _K), jnp.int32),
        pltpu.VMEM((_K, _H), jnp.float32),
        pltpu.VMEM((_K, _H), jnp.float32),
        pltpu.VMEM((_RCH,), jnp.int32),
        pltpu.VMEM((_RCH, _H), jnp.float32),
        pltpu.VMEM_SHARED((_NPAD, _H), jnp.float32),
        pltpu.SemaphoreType.DMA,
        pltpu.SemaphoreType.DMA,
    ],
)
def _sc_aggregate(g_hbm, src_hbm, dst_hbm, out_hbm,
                  src0_v, dst0_v, src1_v, dst1_v, rows0_v, rows1_v,
                  ridx_v, stage_v, acc_sh, sem0, sem1):
    c = lax.axis_index("c")
    s = lax.axis_index("s")
    w = c * _NS + s

    def fill_zero(i, carry):
        for j in range(_H // 16):
            stage_v[i, pl.ds(j * 16, 16)] = jnp.zeros((16,), jnp.float32)
        return carry

    lax.fori_loop(0, _RCH, fill_zero, 0)

    base_row = s * _RPW
    for t in range(_NCH):
        def fill_idx(i, carry):
            ridx_v[pl.ds(i * 16, 16)] = (base_row + t * _RCH + i * 16
                                         + lax.iota(jnp.int32, 16))
            return carry

        lax.fori_loop(0, _RCH // 16, fill_idx, 0)
        pltpu.sync_copy(stage_v, acc_sh.at[ridx_v])
    plsc.subcore_barrier()

    bufs = ((src0_v, dst0_v, rows0_v, sem0), (src1_v, dst1_v, rows1_v, sem1))

    pltpu.sync_copy(src_hbm.at[w, 0], src0_v.at[0])
    pltpu.sync_copy(dst_hbm.at[w, 0], dst0_v.at[0])
    pltpu.async_copy(g_hbm.at[src0_v.at[0]], rows0_v, sem0)

    def _step(i, cur, nxt):
        csrc, cdst, crows, csem = cur
        nsrc, ndst, nrows, nsem = nxt

        @pl.when(i + 1 < _NBATCH)
        def _():
            pltpu.sync_copy(src_hbm.at[w, i + 1], nsrc.at[0])
            pltpu.sync_copy(dst_hbm.at[w, i + 1], ndst.at[0])
            pltpu.async_copy(g_hbm.at[nsrc.at[0]], nrows, nsem)

        pltpu.make_async_copy(g_hbm.at[csrc.at[0]], crows, csem).wait()
        pltpu.sync_copy(crows, acc_sh.at[cdst.at[0]], add=True)

    def body(i, carry):
        @pl.when(i % 2 == 0)
        def _():
            _step(i, bufs[0], bufs[1])

        @pl.when(i % 2 == 1)
        def _():
            _step(i, bufs[1], bufs[0])

        return carry

    lax.fori_loop(0, _NBATCH, body, 0)
    plsc.subcore_barrier()

    for t in range(_NCH):
        def fill_idx2(i, carry):
            ridx_v[pl.ds(i * 16, 16)] = (base_row + t * _RCH + i * 16
                                         + lax.iota(jnp.int32, 16))
            return carry

        lax.fori_loop(0, _RCH // 16, fill_idx2, 0)
        pltpu.async_copy(acc_sh.at[ridx_v], stage_v, sem0).wait()
        pltpu.sync_copy(stage_v, out_hbm.at[c, pl.ds(base_row + t * _RCH, _RCH)])


def _stage0_body(x_ref, w1_ref, degp_ref, batch_ref, g_ref, inv_ref, cnt_ref):
    deg = degp_ref[0, :, :1] + degp_ref[1, :, :1]
    inv = lax.rsqrt(jnp.maximum(deg, 1.0))
    inv_ref[...] = jnp.broadcast_to(inv, (_BLK, 16))
    g_ref[...] = jnp.dot(x_ref[...], w1_ref[...],
                         preferred_element_type=jnp.float32) * inv
    iota = lax.broadcasted_iota(jnp.int32, (1, _B), 1)
    onehot = (batch_ref[...] == iota).astype(jnp.float32)
    cnt = jnp.sum(onehot, axis=0)

    @pl.when(pl.program_id(0) == 0)
    def _():
        cnt_ref[...] = jnp.zeros((_B, _H), jnp.float32)

    cnt_ref[...] += jnp.broadcast_to(cnt[:, None], (_B, _H))


_stage0 = pl.pallas_call(
    _stage0_body,
    grid=(_NBLK,),
    in_specs=[
        pl.BlockSpec((_BLK, _H), lambda i: (i, 0)),
        pl.BlockSpec((_H, _H), lambda i: (0, 0)),
        pl.BlockSpec((_NC, _BLK, _H), lambda i: (0, i, 0)),
        pl.BlockSpec((_BLK, 1), lambda i: (i, 0)),
    ],
    out_specs=[
        pl.BlockSpec((_BLK, _H), lambda i: (i, 0)),
        pl.BlockSpec((_BLK, 16), lambda i: (i, 0)),
        pl.BlockSpec((_B, _H), lambda i: (0, 0)),
    ],
    out_shape=[
        jax.ShapeDtypeStruct((_N, _H), jnp.float32),
        jax.ShapeDtypeStruct((_N, 16), jnp.float32),
        jax.ShapeDtypeStruct((_B, _H), jnp.float32),
    ],
)


def _layer_body(with_next, *refs):
    if with_next:
        (parts_ref, inv_ref, b_ref, s_ref, batch_ref, wn_ref,
         g_ref, gmp_ref, gsum_ref, scal_ref) = refs
    else:
        (parts_ref, inv_ref, b_ref, s_ref, batch_ref,
         gmp_ref, gsum_ref, scal_ref) = refs

    inv = inv_ref[:, :1]
    h = parts_ref[0] + parts_ref[1]
    h = jnp.maximum(h * inv + b_ref[...], 0.0)
    z = jnp.sum(h * s_ref[...], axis=1, keepdims=True)
    p = jax.nn.sigmoid(z)
    pc = jnp.clip(p, 1e-6, 1.0 - 1e-6)
    klp = jnp.sum(pc * jnp.log(2.0 * pc)
                  + (1.0 - pc) * jnp.log(2.0 * (1.0 - pc)))
    refp = jnp.sum((pc - 0.5) ** 2)
    hp = h * p

    if with_next:
        g_ref[...] = jnp.dot(hp, wn_ref[...],
                             preferred_element_type=jnp.float32) * inv

    iota = lax.broadcasted_iota(jnp.int32, (1, _B), 1)
    onehot = (batch_ref[...] == iota).astype(jnp.float32)
    gsum_blk = lax.dot_general(onehot, hp, (((0,), (0,)), ((), ())),
                               preferred_element_type=jnp.float32)

    rows = []
    for b in range(_B):
        mask = (batch_ref[...] == b).astype(jnp.float32)
        rows.append(jnp.max(hp * mask, axis=0, keepdims=True))
    gmp_blk = jnp.concatenate(rows, axis=0)

    rowidx = lax.broadcasted_iota(jnp.int32, (8, _H), 0)
    scal_blk = (jnp.where(rowidx == 0, klp, 0.0)
                + jnp.where(rowidx == 1, refp, 0.0))

    @pl.when(pl.program_id(0) == 0)
    def _():
        gmp_ref[...] = jnp.zeros((_B, _H), jnp.float32)
        gsum_ref[...] = jnp.zeros((_B, _H), jnp.float32)
        scal_ref[...] = jnp.zeros((8, _H), jnp.float32)

    gmp_ref[...] = jnp.maximum(gmp_ref[...], gmp_blk)
    gsum_ref[...] += gsum_blk
    scal_ref[...] += scal_blk


_layer_in_specs = [
    pl.BlockSpec((_NC, _BLK, _H), lambda i: (0, i, 0)),
    pl.BlockSpec((_BLK, 16), lambda i: (i, 0)),
    pl.BlockSpec((1, _H), lambda i: (0, 0)),
    pl.BlockSpec((1, _H), lambda i: (0, 0)),
    pl.BlockSpec((_BLK, 1), lambda i: (i, 0)),
]
_readout_out_specs = [
    pl.BlockSpec((_B, _H), lambda i: (0, 0)),
    pl.BlockSpec((_B, _H), lambda i: (0, 0)),
    pl.BlockSpec((8, _H), lambda i: (0, 0)),
]
_readout_out_shape = [
    jax.ShapeDtypeStruct((_B, _H), jnp.float32),
    jax.ShapeDtypeStruct((_B, _H), jnp.float32),
    jax.ShapeDtypeStruct((8, _H), jnp.float32),
]

_layer_mid = pl.pallas_call(
    functools.partial(_layer_body, True),
    grid=(_NBLK,),
    in_specs=_layer_in_specs + [pl.BlockSpec((_H, _H), lambda i: (0, 0))],
    out_specs=[pl.BlockSpec((_BLK, _H), lambda i: (i, 0))] + _readout_out_specs,
    out_shape=[jax.ShapeDtypeStruct((_N, _H), jnp.float32)] + _readout_out_shape,
)

_layer_last = pl.pallas_call(
    functools.partial(_layer_body, False),
    grid=(_NBLK,),
    in_specs=_layer_in_specs,
    out_specs=_readout_out_specs,
    out_shape=_readout_out_shape,
)


def _final_body(gmp1, gsum1, gmp2, gsum2, gmp3, gsum3, cnt,
                wl1, bl1, wl2, bl2, wl3, bl3, out_ref):
    cntc = jnp.maximum(cnt[...], 1.0)

    def xin(gmp, gsum):
        return jnp.concatenate([gmp[...], gsum[...] / cntc], axis=1)

    g = (jnp.maximum(xin(gmp1, gsum1), 0.0)
         + jnp.maximum(xin(gmp2, gsum2), 0.0)
         + jnp.maximum(xin(gmp3, gsum3), 0.0))
    g = jnp.maximum(jnp.dot(g, wl1[...],
                            preferred_element_type=jnp.float32) + bl1[...], 0.0)
    g = jnp.maximum(jnp.dot(g, wl2[...],
                            preferred_element_type=jnp.float32) + bl2[...], 0.0)
    z = jnp.dot(g, wl3[...], preferred_element_type=jnp.float32) + bl3[...]
    lane = lax.broadcasted_iota(jnp.int32, (_B, _H), 1)
    zm = jnp.where(lane < _C, z, -1e30)
    mx = jnp.max(zm, axis=1, keepdims=True)
    lse = mx + jnp.log(jnp.sum(jnp.exp(zm - mx), axis=1, keepdims=True))
    out_ref[...] = zm - lse


_final = pl.pallas_call(
    _final_body,
    grid=(1,),
    in_specs=[pl.BlockSpec((_B, _H), lambda i: (0, 0))] * 7
    + [
        pl.BlockSpec((2 * _H, _H), lambda i: (0, 0)),
        pl.BlockSpec((1, _H), lambda i: (0, 0)),
        pl.BlockSpec((_H, _H), lambda i: (0, 0)),
        pl.BlockSpec((1, _H), lambda i: (0, 0)),
        pl.BlockSpec((_H, _H), lambda i: (0, 0)),
        pl.BlockSpec((1, _H), lambda i: (0, 0)),
    ],
    out_specs=pl.BlockSpec((_B, _H), lambda i: (0, 0)),
    out_shape=jax.ShapeDtypeStruct((_B, _H), jnp.float32),
)


def kernel(x, edge_index, batch, W1, b1, W2, b2, W3, b3,
           s1, s2, s3, Wl1, bl1, Wl2, bl2, Wl3, bl3):
    src3 = edge_index[0].reshape(_NW, _NBATCH, _K)
    dst3 = edge_index[1].reshape(_NW, _NBATCH, _K)
    batch2 = batch.reshape(_N, 1)

    degp = _sc_degree(dst3)
    g1, inv16, cnt = _stage0(x, W1, degp, batch2)

    p1 = _sc_aggregate(g1, src3, dst3)
    g2, gmp1, gsum1, sc1 = _layer_mid(p1, inv16, b1.reshape(1, _H),
                                      s1.reshape(1, _H), batch2, W2)
    p2 = _sc_aggregate(g2, src3, dst3)
    g3, gmp2, gsum2, sc2 = _layer_mid(p2, inv16, b2.reshape(1, _H),
                                      s2.reshape(1, _H), batch2, W3)
    p3 = _sc_aggregate(g3, src3, dst3)
    gmp3, gsum3, sc3 = _layer_last(p3, inv16, b3.reshape(1, _H),
                                   s3.reshape(1, _H), batch2)

    Wl2p = jnp.pad(Wl2, ((0, 0), (0, _H - Wl2.shape[1])))
    bl2p = jnp.pad(bl2, (0, _H - bl2.shape[0])).reshape(1, _H)
    Wl3p = jnp.pad(Wl3, ((0, _H - Wl3.shape[0]), (0, _H - Wl3.shape[1])))
    bl3p = jnp.pad(bl3, (0, _H - bl3.shape[0])).reshape(1, _H)

    logits128 = _final(gmp1, gsum1, gmp2, gsum2, gmp3, gsum3, cnt,
                       Wl1, bl1.reshape(1, _H), Wl2p, bl2p, Wl3p, bl3p)
    logits = logits128[:, :_C]
    kl = (sc1[0, 0] + sc2[0, 0] + sc3[0, 0]) / _N
    refv = (sc1[1, 0] + sc2[1, 0] + sc3[1, 0]) / _N
    return (logits, kl, refv)


# speedup vs baseline: 11.0165x; 1.0018x over previous
import functools

import jax
import jax.numpy as jnp
from jax import lax
from jax.experimental import pallas as pl
from jax.experimental.pallas import tpu as pltpu
from jax.experimental.pallas import tpu_sc as plsc

_N = 10000
_E = 320000
_H = 128
_B = 64
_C = 10

_NC = 2
_NS = 16
_NW = _NC * _NS
_EPW = _E // _NW
_K = 80
_NBATCH = _EPW // _K
_NPAD = 10240
_RPW = _NPAD // _NS
_RCH = 128
_NCH = _RPW // _RCH

_BLK = 1000
_NBLK = _N // _BLK

_sc_mesh = plsc.VectorSubcoreMesh(core_axis_name="c", subcore_axis_name="s")


@functools.partial(
    pl.kernel,
    mesh=_sc_mesh,
    out_type=jax.ShapeDtypeStruct((_NC, _NPAD, _H), jnp.float32),
    scratch_types=[
        pltpu.VMEM((_NBATCH, _K), jnp.int32),
        pltpu.VMEM((_K, _H), jnp.float32),
        pltpu.VMEM((_RCH,), jnp.int32),
        pltpu.VMEM((_RCH, _H), jnp.float32),
        pltpu.VMEM_SHARED((_NPAD, _H), jnp.float32),
        pltpu.SemaphoreType.DMA,
    ],
)
def _sc_degree(dst_hbm, out_hbm, dstb_v, ones_v, ridx_v, stage_v, acc_sh, sem):
    c = lax.axis_index("c")
    s = lax.axis_index("s")
    w = c * _NS + s

    def fill_ones(i, carry):
        for j in range(_H // 16):
            ones_v[i, pl.ds(j * 16, 16)] = jnp.full((16,), 1.0, jnp.float32)
        return carry

    lax.fori_loop(0, _K, fill_ones, 0)

    def fill_zero(i, carry):
        for j in range(_H // 16):
            stage_v[i, pl.ds(j * 16, 16)] = jnp.zeros((16,), jnp.float32)
        return carry

    lax.fori_loop(0, _RCH, fill_zero, 0)

    base_row = s * _RPW
    for t in range(_NCH):
        def fill_idx(i, carry):
            ridx_v[pl.ds(i * 16, 16)] = (base_row + t * _RCH + i * 16
                                         + lax.iota(jnp.int32, 16))
            return carry

        lax.fori_loop(0, _RCH // 16, fill_idx, 0)
        pltpu.sync_copy(stage_v, acc_sh.at[ridx_v])
    plsc.subcore_barrier()

    pltpu.sync_copy(dst_hbm.at[w], dstb_v)

    def body(i, carry):
        pltpu.sync_copy(ones_v, acc_sh.at[dstb_v.at[i]], add=True)
        return carry

    lax.fori_loop(0, _NBATCH, body, 0)
    plsc.subcore_barrier()

    for t in range(_NCH):
        def fill_idx2(i, carry):
            ridx_v[pl.ds(i * 16, 16)] = (base_row + t * _RCH + i * 16
                                         + lax.iota(jnp.int32, 16))
            return carry

        lax.fori_loop(0, _RCH // 16, fill_idx2, 0)
        pltpu.async_copy(acc_sh.at[ridx_v], stage_v, sem).wait()
        pltpu.sync_copy(stage_v, out_hbm.at[c, pl.ds(base_row + t * _RCH, _RCH)])


@functools.partial(
    pl.kernel,
    mesh=_sc_mesh,
    out_type=jax.ShapeDtypeStruct((_NC, _NPAD, _H), jnp.float32),
    scratch_types=[
        pltpu.VMEM((1, _K), jnp.int32),
        pltpu.VMEM((1, _K), jnp.int32),
        pltpu.VMEM((1, _K), jnp.int32),
        pltpu.VMEM((1, _K), jnp.int32),
        pltpu.VMEM((_K, _H), jnp.float32),
        pltpu.VMEM((_K, _H), jnp.float32),
        pltpu.VMEM((_RCH,), jnp.int32),
        pltpu.VMEM((_RCH, _H), jnp.float32),
        pltpu.VMEM_SHARED((_NPAD, _H), jnp.float32),
        pltpu.SemaphoreType.DMA,
        pltpu.SemaphoreType.DMA,
        pltpu.SemaphoreType.DMA,
        pltpu.SemaphoreType.DMA,
    ],
)
def _sc_aggregate(g_hbm, src_hbm, dst_hbm, out_hbm,
                  src0_v, dst0_v, src1_v, dst1_v, rows0_v, rows1_v,
                  ridx_v, stage_v, acc_sh, sem0, sem1, ssem0, ssem1):
    c = lax.axis_index("c")
    s = lax.axis_index("s")
    w = c * _NS + s

    def fill_zero(i, carry):
        for j in range(_H // 16):
            stage_v[i, pl.ds(j * 16, 16)] = jnp.zeros((16,), jnp.float32)
        return carry

    lax.fori_loop(0, _RCH, fill_zero, 0)

    base_row = s * _RPW
    for t in range(_NCH):
        def fill_idx(i, carry):
            ridx_v[pl.ds(i * 16, 16)] = (base_row + t * _RCH + i * 16
                                         + lax.iota(jnp.int32, 16))
            return carry

        lax.fori_loop(0, _RCH // 16, fill_idx, 0)
        pltpu.sync_copy(stage_v, acc_sh.at[ridx_v])
    plsc.subcore_barrier()

    bufs = ((src0_v, dst0_v, rows0_v, sem0, ssem0),
            (src1_v, dst1_v, rows1_v, sem1, ssem1))

    pltpu.sync_copy(src_hbm.at[w, 0], src0_v.at[0])
    pltpu.sync_copy(dst_hbm.at[w, 0], dst0_v.at[0])
    pltpu.async_copy(g_hbm.at[src0_v.at[0]], rows0_v, sem0)

    def _step(i, cur, nxt):
        csrc, cdst, crows, csem, cssem = cur
        nsrc, ndst, nrows, nsem, nssem = nxt

        @pl.when(i + 1 < _NBATCH)
        def _():
            @pl.when(i > 0)
            def _():
                pltpu.make_async_copy(nrows, acc_sh.at[ndst.at[0]],
                                      nssem).wait()

            pltpu.sync_copy(src_hbm.at[w, i + 1], nsrc.at[0])
            pltpu.sync_copy(dst_hbm.at[w, i + 1], ndst.at[0])
            pltpu.async_copy(g_hbm.at[nsrc.at[0]], nrows, nsem)

        pltpu.make_async_copy(g_hbm.at[csrc.at[0]], crows, csem).wait()
        pltpu.async_copy(crows, acc_sh.at[cdst.at[0]], cssem, add=True)

    def body(i, carry):
        @pl.when(i % 2 == 0)
        def _():
            _step(i, bufs[0], bufs[1])

        @pl.when(i % 2 == 1)
        def _():
            _step(i, bufs[1], bufs[0])

        return carry

    lax.fori_loop(0, _NBATCH, body, 0)
    pltpu.make_async_copy(rows1_v, acc_sh.at[dst1_v.at[0]], ssem1).wait()
    pltpu.make_async_copy(rows0_v, acc_sh.at[dst0_v.at[0]], ssem0).wait()
    plsc.subcore_barrier()

    for t in range(_NCH):
        def fill_idx2(i, carry):
            ridx_v[pl.ds(i * 16, 16)] = (base_row + t * _RCH + i * 16
                                         + lax.iota(jnp.int32, 16))
            return carry

        lax.fori_loop(0, _RCH // 16, fill_idx2, 0)
        pltpu.async_copy(acc_sh.at[ridx_v], stage_v, sem0).wait()
        pltpu.sync_copy(stage_v, out_hbm.at[c, pl.ds(base_row + t * _RCH, _RCH)])


def _stage0_body(x_ref, w1_ref, degp_ref, batch_ref, g_ref, inv_ref, cnt_ref):
    deg = degp_ref[0, :, :1] + degp_ref[1, :, :1]
    inv = lax.rsqrt(jnp.maximum(deg, 1.0))
    inv_ref[...] = jnp.broadcast_to(inv, (_BLK, 16))
    g_ref[...] = jnp.dot(x_ref[...], w1_ref[...],
                         preferred_element_type=jnp.float32) * inv
    iota = lax.broadcasted_iota(jnp.int32, (1, _B), 1)
    onehot = (batch_ref[...] == iota).astype(jnp.float32)
    cnt = jnp.sum(onehot, axis=0)

    @pl.when(pl.program_id(0) == 0)
    def _():
        cnt_ref[...] = jnp.zeros((_B, _H), jnp.float32)

    cnt_ref[...] += jnp.broadcast_to(cnt[:, None], (_B, _H))


_stage0 = pl.pallas_call(
    _stage0_body,
    grid=(_NBLK,),
    in_specs=[
        pl.BlockSpec((_BLK, _H), lambda i: (i, 0)),
        pl.BlockSpec((_H, _H), lambda i: (0, 0)),
        pl.BlockSpec((_NC, _BLK, _H), lambda i: (0, i, 0)),
        pl.BlockSpec((_BLK, 1), lambda i: (i, 0)),
    ],
    out_specs=[
        pl.BlockSpec((_BLK, _H), lambda i: (i, 0)),
        pl.BlockSpec((_BLK, 16), lambda i: (i, 0)),
        pl.BlockSpec((_B, _H), lambda i: (0, 0)),
    ],
    out_shape=[
        jax.ShapeDtypeStruct((_N, _H), jnp.float32),
        jax.ShapeDtypeStruct((_N, 16), jnp.float32),
        jax.ShapeDtypeStruct((_B, _H), jnp.float32),
    ],
)


def _layer_body(with_next, *refs):
    if with_next:
        (parts_ref, inv_ref, b_ref, s_ref, batch_ref, wn_ref,
         g_ref, gmp_ref, gsum_ref, scal_ref) = refs
    else:
        (parts_ref, inv_ref, b_ref, s_ref, batch_ref,
         gmp_ref, gsum_ref, scal_ref) = refs

    inv = inv_ref[:, :1]
    h = parts_ref[0] + parts_ref[1]
    h = jnp.maximum(h * inv + b_ref[...], 0.0)
    z = jnp.sum(h * s_ref[...], axis=1, keepdims=True)
    p = jax.nn.sigmoid(z)
    pc = jnp.clip(p, 1e-6, 1.0 - 1e-6)
    klp = jnp.sum(pc * jnp.log(2.0 * pc)
                  + (1.0 - pc) * jnp.log(2.0 * (1.0 - pc)))
    refp = jnp.sum((pc - 0.5) ** 2)
    hp = h * p

    if with_next:
        g_ref[...] = jnp.dot(hp, wn_ref[...],
                             preferred_element_type=jnp.float32) * inv

    iota = lax.broadcasted_iota(jnp.int32, (1, _B), 1)
    onehot = (batch_ref[...] == iota).astype(jnp.float32)
    gsum_blk = lax.dot_general(onehot, hp, (((0,), (0,)), ((), ())),
                               preferred_element_type=jnp.float32)

    rows = []
    for b in range(_B):
        mask = (batch_ref[...] == b).astype(jnp.float32)
        rows.append(jnp.max(hp * mask, axis=0, keepdims=True))
    gmp_blk = jnp.concatenate(rows, axis=0)

    rowidx = lax.broadcasted_iota(jnp.int32, (8, _H), 0)
    scal_blk = (jnp.where(rowidx == 0, klp, 0.0)
                + jnp.where(rowidx == 1, refp, 0.0))

    @pl.when(pl.program_id(0) == 0)
    def _():
        gmp_ref[...] = jnp.zeros((_B, _H), jnp.float32)
        gsum_ref[...] = jnp.zeros((_B, _H), jnp.float32)
        scal_ref[...] = jnp.zeros((8, _H), jnp.float32)

    gmp_ref[...] = jnp.maximum(gmp_ref[...], gmp_blk)
    gsum_ref[...] += gsum_blk
    scal_ref[...] += scal_blk


_layer_in_specs = [
    pl.BlockSpec((_NC, _BLK, _H), lambda i: (0, i, 0)),
    pl.BlockSpec((_BLK, 16), lambda i: (i, 0)),
    pl.BlockSpec((1, _H), lambda i: (0, 0)),
    pl.BlockSpec((1, _H), lambda i: (0, 0)),
    pl.BlockSpec((_BLK, 1), lambda i: (i, 0)),
]
_readout_out_specs = [
    pl.BlockSpec((_B, _H), lambda i: (0, 0)),
    pl.BlockSpec((_B, _H), lambda i: (0, 0)),
    pl.BlockSpec((8, _H), lambda i: (0, 0)),
]
_readout_out_shape = [
    jax.ShapeDtypeStruct((_B, _H), jnp.float32),
    jax.ShapeDtypeStruct((_B, _H), jnp.float32),
    jax.ShapeDtypeStruct((8, _H), jnp.float32),
]

_layer_mid = pl.pallas_call(
    functools.partial(_layer_body, True),
    grid=(_NBLK,),
    in_specs=_layer_in_specs + [pl.BlockSpec((_H, _H), lambda i: (0, 0))],
    out_specs=[pl.BlockSpec((_BLK, _H), lambda i: (i, 0))] + _readout_out_specs,
    out_shape=[jax.ShapeDtypeStruct((_N, _H), jnp.float32)] + _readout_out_shape,
)

_layer_last = pl.pallas_call(
    functools.partial(_layer_body, False),
    grid=(_NBLK,),
    in_specs=_layer_in_specs,
    out_specs=_readout_out_specs,
    out_shape=_readout_out_shape,
)


def _final_body(gmp1, gsum1, gmp2, gsum2, gmp3, gsum3, cnt,
                wl1, bl1, wl2, bl2, wl3, bl3, out_ref):
    cntc = jnp.maximum(cnt[...], 1.0)

    def xin(gmp, gsum):
        return jnp.concatenate([gmp[...], gsum[...] / cntc], axis=1)

    g = (jnp.maximum(xin(gmp1, gsum1), 0.0)
         + jnp.maximum(xin(gmp2, gsum2), 0.0)
         + jnp.maximum(xin(gmp3, gsum3), 0.0))
    g = jnp.maximum(jnp.dot(g, wl1[...],
                            preferred_element_type=jnp.float32) + bl1[...], 0.0)
    g = jnp.maximum(jnp.dot(g, wl2[...],
                            preferred_element_type=jnp.float32) + bl2[...], 0.0)
    z = jnp.dot(g, wl3[...], preferred_element_type=jnp.float32) + bl3[...]
    lane = lax.broadcasted_iota(jnp.int32, (_B, _H), 1)
    zm = jnp.where(lane < _C, z, -1e30)
    mx = jnp.max(zm, axis=1, keepdims=True)
    lse = mx + jnp.log(jnp.sum(jnp.exp(zm - mx), axis=1, keepdims=True))
    out_ref[...] = zm - lse


_final = pl.pallas_call(
    _final_body,
    grid=(1,),
    in_specs=[pl.BlockSpec((_B, _H), lambda i: (0, 0))] * 7
    + [
        pl.BlockSpec((2 * _H, _H), lambda i: (0, 0)),
        pl.BlockSpec((1, _H), lambda i: (0, 0)),
        pl.BlockSpec((_H, _H), lambda i: (0, 0)),
        pl.BlockSpec((1, _H), lambda i: (0, 0)),
        pl.BlockSpec((_H, _H), lambda i: (0, 0)),
        pl.BlockSpec((1, _H), lambda i: (0, 0)),
    ],
    out_specs=pl.BlockSpec((_B, _H), lambda i: (0, 0)),
    out_shape=jax.ShapeDtypeStruct((_B, _H), jnp.float32),
)


def kernel(x, edge_index, batch, W1, b1, W2, b2, W3, b3,
           s1, s2, s3, Wl1, bl1, Wl2, bl2, Wl3, bl3):
    src3 = edge_index[0].reshape(_NW, _NBATCH, _K)
    dst3 = edge_index[1].reshape(_NW, _NBATCH, _K)
    batch2 = batch.reshape(_N, 1)

    degp = _sc_degree(dst3)
    g1, inv16, cnt = _stage0(x, W1, degp, batch2)

    p1 = _sc_aggregate(g1, src3, dst3)
    g2, gmp1, gsum1, sc1 = _layer_mid(p1, inv16, b1.reshape(1, _H),
                                      s1.reshape(1, _H), batch2, W2)
    p2 = _sc_aggregate(g2, src3, dst3)
    g3, gmp2, gsum2, sc2 = _layer_mid(p2, inv16, b2.reshape(1, _H),
                                      s2.reshape(1, _H), batch2, W3)
    p3 = _sc_aggregate(g3, src3, dst3)
    gmp3, gsum3, sc3 = _layer_last(p3, inv16, b3.reshape(1, _H),
                                   s3.reshape(1, _H), batch2)

    Wl2p = jnp.pad(Wl2, ((0, 0), (0, _H - Wl2.shape[1])))
    bl2p = jnp.pad(bl2, (0, _H - bl2.shape[0])).reshape(1, _H)
    Wl3p = jnp.pad(Wl3, ((0, _H - Wl3.shape[0]), (0, _H - Wl3.shape[1])))
    bl3p = jnp.pad(bl3, (0, _H - bl3.shape[0])).reshape(1, _H)

    logits128 = _final(gmp1, gsum1, gmp2, gsum2, gmp3, gsum3, cnt,
                       Wl1, bl1.reshape(1, _H), Wl2p, bl2p, Wl3p, bl3p)
    logits = logits128[:, :_C]
    kl = (sc1[0, 0] + sc2[0, 0] + sc3[0, 0]) / _N
    refv = (sc1[1, 0] + sc2[1, 0] + sc3[1, 0]) / _N
    return (logits, kl, refv)
